# Initial kernel scaffold; baseline (speedup 1.0000x reference)
import jax
import jax.numpy as jnp
from jax.experimental import pallas as pl


def kernel(x, edge_index_rel0, edge_index_rel1, edge_index_rel2, W_rel0, W_rel1, W_rel2):
    raise NotImplementedError("write your pallas kernel here")


# speedup vs baseline: 6.5441x; 6.5441x over previous
import functools

import jax
import jax.numpy as jnp
from jax import lax
from jax.experimental import pallas as pl
from jax.experimental.pallas import tpu as pltpu
from jax.experimental.pallas import tpu_sc as plsc

N = 10000
E = 320000
D = 128
R = 3

CHUNK = 128
NW = 32
CH = 236
TOT_E = NW * CH * CHUNK
PAD_E = TOT_E - R * E
PADN = 10240
DUMMY = N
ZR = 64
BM = 1000


def _mm_body(x_ref, w_ref, o_ref):
    o_ref[0] = jnp.dot(x_ref[...], w_ref[0], preferred_element_type=jnp.float32)


def _project(x, Ws):
    return pl.pallas_call(
        _mm_body,
        grid=(R, N // BM),
        in_specs=[
            pl.BlockSpec((BM, D), lambda r, i: (i, 0)),
            pl.BlockSpec((1, D, D), lambda r, i: (r, 0, 0)),
        ],
        out_specs=pl.BlockSpec((1, BM, D), lambda r, i: (r, i, 0)),
        out_shape=jax.ShapeDtypeStruct((R, N, D), jnp.float32),
    )(x, Ws)


def _comb_body(p_ref, o_ref):
    o_ref[...] = jnp.maximum(p_ref[0] + p_ref[1], 0.0)


def _combine(partials):
    return pl.pallas_call(
        _comb_body,
        grid=(N // BM,),
        in_specs=[pl.BlockSpec((2, BM, D), lambda i: (0, i, 0))],
        out_specs=pl.BlockSpec((BM, D), lambda i: (i, 0)),
        out_shape=jax.ShapeDtypeStruct((N, D), jnp.float32),
    )(partials)


_MESH = plsc.VectorSubcoreMesh(core_axis_name="c", subcore_axis_name="s")


@functools.partial(
    pl.kernel,
    out_type=jax.ShapeDtypeStruct((2, PADN, D), jnp.float32),
    mesh=_MESH,
    scratch_types=[
        pltpu.VMEM((CHUNK,), jnp.int32),
        pltpu.VMEM((CHUNK,), jnp.int32),
        pltpu.VMEM((CHUNK,), jnp.int32),
        pltpu.VMEM((CHUNK,), jnp.int32),
        pltpu.VMEM((CHUNK,), jnp.int32),
        pltpu.VMEM((CHUNK,), jnp.int32),
        pltpu.VMEM((CHUNK,), jnp.int32),
        pltpu.VMEM((CHUNK,), jnp.int32),
        pltpu.VMEM((CHUNK, D), jnp.float32),
        pltpu.VMEM((CHUNK, D), jnp.float32),
        pltpu.VMEM((ZR, D), jnp.float32),
        pltpu.VMEM_SHARED((PADN, D), jnp.float32),
        pltpu.SemaphoreType.DMA,
        pltpu.SemaphoreType.DMA,
        pltpu.SemaphoreType.DMA,
        pltpu.SemaphoreType.DMA,
        pltpu.SemaphoreType.DMA,
        pltpu.SemaphoreType.DMA,
    ],
)
def _sc_edge(src_hbm, dst_hbm, h_hbm, out_hbm,
             sv0, sv1, sv2, sv3, dv0, dv1, dv2, dv3,
             rows0, rows1, zbuf, acc,
             g0, g1, i0, i1, i2, i3):
    cid = lax.axis_index("c")
    sid = lax.axis_index("s")
    wid = sid * 2 + cid
    base_e = wid * (CH * CHUNK)

    svs = (sv0, sv1, sv2, sv3)
    dvs = (dv0, dv1, dv2, dv3)
    rows = (rows0, rows1)
    gsems = (g0, g1)
    isems = (i0, i1, i2, i3)

    zz = jnp.zeros((16,), jnp.float32)
    for i in range(ZR):
        for j in range(D // 16):
            zbuf[i, pl.ds(j * 16, 16)] = zz
    rows_per_tile = PADN // 16
    for t in range(rows_per_tile // ZR):
        pltpu.sync_copy(zbuf, acc.at[pl.ds(sid * rows_per_tile + t * ZR, ZR)])
    plsc.subcore_barrier()

    def issue_idx(b, j):
        off = base_e + j * CHUNK
        pltpu.async_copy(src_hbm.at[pl.ds(off, CHUNK)], svs[b], isems[b])
        pltpu.async_copy(dst_hbm.at[pl.ds(off, CHUNK)], dvs[b], isems[b])

    def wait_idx(b):
        pltpu.make_async_copy(src_hbm.at[pl.ds(0, CHUNK)], svs[b], isems[b]).wait()
        pltpu.make_async_copy(dst_hbm.at[pl.ds(0, CHUNK)], dvs[b], isems[b]).wait()

    def issue_gather(rb, b):
        pltpu.async_copy(h_hbm.at[svs[b]], rows[rb], gsems[rb])

    def wait_gather(rb, b):
        pltpu.make_async_copy(h_hbm.at[svs[b]], rows[rb], gsems[rb]).wait()

    for b in range(4):
        issue_idx(b, b)
    wait_idx(0)
    issue_gather(0, 0)
    wait_idx(1)
    issue_gather(1, 1)

    def outer(g, _):
        for b4 in range(4):
            j = g * 4 + b4
            rb = b4 % 2
            b2 = (b4 + 2) % 4
            wait_gather(rb, b4)
            pltpu.sync_copy(rows[rb], acc.at[dvs[b4]], add=True)

            @pl.when(j + 2 < CH)
            def _():
                wait_idx(b2)
                issue_gather(rb, b2)

            @pl.when(j + 4 < CH)
            def _():
                issue_idx(b4, j + 4)

        return 0

    lax.fori_loop(0, CH // 4, outer, 0)

    plsc.subcore_barrier()
    out_rows = PADN // 16
    pltpu.sync_copy(
        acc.at[pl.ds(sid * out_rows, out_rows)],
        out_hbm.at[cid, pl.ds(sid * out_rows, out_rows), :],
    )


def kernel(x, edge_index_rel0, edge_index_rel1, edge_index_rel2,
           W_rel0, W_rel1, W_rel2):
    Ws = jnp.stack([W_rel0, W_rel1, W_rel2])
    h = _project(x, Ws).reshape(R * N, D)

    pad_src = jnp.zeros((PAD_E,), jnp.int32)
    pad_dst = jnp.full((PAD_E,), DUMMY, jnp.int32)
    src = jnp.concatenate(
        [edge_index_rel0[0], edge_index_rel1[0] + N, edge_index_rel2[0] + 2 * N,
         pad_src])
    dst = jnp.concatenate(
        [edge_index_rel0[1], edge_index_rel1[1], edge_index_rel2[1], pad_dst])

    partials = _sc_edge(src, dst, h)
    return _combine(partials)
